# baseline (speedup 1.0000x reference)
import functools

import jax
import jax.numpy as jnp
from jax import lax
from jax.experimental import pallas as pl
from jax.experimental.pallas import tpu as pltpu
from jax.experimental.pallas import tpu_sc as plsc

_WIDE = 128
_CHUNK = 128


def _make_widen(vocab, dim):
    info = plsc.get_sparse_core_info()
    num_workers = info.num_cores * info.num_subcores
    rows_per_w = vocab // num_workers
    mesh = plsc.VectorSubcoreMesh(core_axis_name="c", subcore_axis_name="s")

    @functools.partial(
        pl.kernel,
        mesh=mesh,
        out_type=jax.ShapeDtypeStruct((vocab, _WIDE), jnp.float32),
        scratch_types=[pltpu.SemaphoreType.DMA],
        compiler_params=pltpu.CompilerParams(needs_layout_passes=False),
    )
    def widen_kernel(table_hbm, wide_hbm, sem):
        wid = lax.axis_index("s") * info.num_cores + lax.axis_index("c")
        chunk = ((rows_per_w + 7) // 8) * 8
        base = pl.multiple_of(
            jnp.minimum(wid * chunk, vocab - chunk), 8
        )
        pltpu.async_copy(
            table_hbm.at[pl.ds(base, chunk)],
            wide_hbm.at[pl.ds(base, chunk), pl.ds(0, dim)],
            sem,
        ).wait()

    return widen_kernel


def _make_gather(batch):
    info = plsc.get_sparse_core_info()
    num_workers = info.num_cores * info.num_subcores
    b_per_w = batch // num_workers
    n_chunks = b_per_w // _CHUNK
    mesh = plsc.VectorSubcoreMesh(core_axis_name="c", subcore_axis_name="s")

    @functools.partial(
        pl.kernel,
        mesh=mesh,
        out_type=jax.ShapeDtypeStruct((batch, _WIDE), jnp.float32),
        scratch_types=[
            pltpu.VMEM((n_chunks, _CHUNK), jnp.int32),
            pltpu.VMEM((_CHUNK, _WIDE), jnp.float32),
            pltpu.VMEM((_CHUNK, _WIDE), jnp.float32),
            pltpu.SemaphoreType.DMA,
            pltpu.SemaphoreType.DMA,
            pltpu.SemaphoreType.DMA,
            pltpu.SemaphoreType.DMA,
        ],
        compiler_params=pltpu.CompilerParams(needs_layout_passes=False),
    )
    def gather_kernel(
        wide_hbm, idx_hbm, out_hbm,
        idx_v, vbuf0, vbuf1, sem_g0, sem_g1, sem_o0, sem_o1,
    ):
        wid = lax.axis_index("s") * info.num_cores + lax.axis_index("c")
        base = wid * b_per_w
        pltpu.sync_copy(idx_hbm.at[pl.ds(wid * n_chunks, n_chunks)], idx_v)

        vbufs = (vbuf0, vbuf1)
        sems_g = (sem_g0, sem_g1)
        sems_o = (sem_o0, sem_o1)

        def launch(c, slot):
            pltpu.async_copy(
                wide_hbm.at[idx_v.at[c]], vbufs[slot], sems_g[slot]
            )

        def drain_gather(slot):
            pltpu.make_async_copy(
                wide_hbm.at[idx_v.at[0]], vbufs[slot], sems_g[slot]
            ).wait()

        def drain_out(slot):
            pltpu.make_async_copy(
                vbufs[slot], out_hbm.at[pl.ds(0, _CHUNK)], sems_o[slot]
            ).wait()

        launch(0, 0)
        launch(1, 1)

        def body(c2):
            for slot in range(2):
                c = 2 * c2 + slot
                drain_gather(slot)

                @pl.when(c >= 2)
                def _():
                    drain_out(slot)

                pltpu.async_copy(
                    vbufs[slot],
                    out_hbm.at[pl.ds(pl.multiple_of(base + c * _CHUNK, 8), _CHUNK)],
                    sems_o[slot],
                )

                @pl.when(c + 2 < n_chunks)
                def _():
                    launch(c + 2, slot)

        pl.loop(0, n_chunks // 2)(body)
        drain_out(0)
        drain_out(1)

    return gather_kernel


def kernel(variable_hash, embedding_table):
    batch = variable_hash.shape[0]
    vocab, dim = embedding_table.shape
    gather = _make_gather(batch)
    wide = jnp.pad(embedding_table, ((0, 0), (0, _WIDE - dim)))
    idx2d = variable_hash.reshape(batch // _CHUNK, _CHUNK)
    out_wide = gather(wide, idx2d)
    return out_wide[:, :dim]


# speedup vs baseline: 1.2311x; 1.2311x over previous
import functools

import jax
import jax.numpy as jnp
from jax import lax
from jax.experimental import pallas as pl
from jax.experimental.pallas import tpu as pltpu
from jax.experimental.pallas import tpu_sc as plsc

_LANES = 16
_STAGED_FRAC_NUM, _STAGED_FRAC_DEN = 5, 8


def _make_gather(batch, dim):
    info = plsc.get_sparse_core_info()
    num_workers = info.num_cores * info.num_subcores
    b_per_w = batch // num_workers
    n_bursts = b_per_w // _LANES
    n_staged = (n_bursts * _STAGED_FRAC_NUM) // _STAGED_FRAC_DEN
    mesh = plsc.VectorSubcoreMesh(core_axis_name="c", subcore_axis_name="s")

    @functools.partial(
        pl.kernel,
        mesh=mesh,
        out_type=jax.ShapeDtypeStruct((batch, dim), jnp.float32),
        scratch_types=[
            pltpu.VMEM((b_per_w,), jnp.int32),
            pltpu.VMEM((b_per_w, dim), jnp.float32),
        ]
        + [pltpu.SemaphoreType.DMA] * 4,
        compiler_params=pltpu.CompilerParams(needs_layout_passes=False),
    )
    def gather_kernel(table_hbm, idx_hbm, out_hbm, idx_v, rows_v, *sems):
        wid = lax.axis_index("s") * info.num_cores + lax.axis_index("c")
        base = wid * b_per_w
        pltpu.sync_copy(idx_hbm.at[pl.ds(base, b_per_w)], idx_v)

        lane_ids = lax.iota(jnp.int32, _LANES)
        neg = jnp.full((_LANES,), jnp.iinfo(jnp.int32).min, jnp.int32)

        def lane_row(v, l):
            return lax.reduce_max(jnp.where(lane_ids == l, v, neg), axes=(0,))

        def burst_staged(k):
            v = idx_v[pl.ds(k * _LANES, _LANES)]
            for l in range(_LANES):
                pltpu.async_copy(
                    table_hbm.at[pl.ds(lane_row(v, l), 1)],
                    rows_v.at[pl.ds(k * _LANES + l, 1)],
                    sems[l % 2],
                )

        def burst_direct(k):
            v = idx_v[pl.ds(k * _LANES, _LANES)]
            for l in range(_LANES):
                pltpu.async_copy(
                    table_hbm.at[pl.ds(lane_row(v, l), 1)],
                    out_hbm.at[pl.ds(base + k * _LANES + l, 1)],
                    sems[2 + l % 2],
                )

        pl.loop(0, n_staged)(burst_staged)
        pl.loop(n_staged, n_bursts)(burst_direct)

        def drain_staged(_):
            for l in range(_LANES):
                pltpu.make_async_copy(
                    table_hbm.at[pl.ds(0, 1)],
                    rows_v.at[pl.ds(0, 1)],
                    sems[l % 2],
                ).wait()

        def drain_direct(_):
            for l in range(_LANES):
                pltpu.make_async_copy(
                    table_hbm.at[pl.ds(0, 1)],
                    out_hbm.at[pl.ds(base, 1)],
                    sems[2 + l % 2],
                ).wait()

        pl.loop(0, n_staged)(drain_staged)
        pl.loop(n_staged, n_bursts)(drain_direct)
        pltpu.sync_copy(
            rows_v.at[pl.ds(0, n_staged * _LANES)],
            out_hbm.at[pl.ds(base, n_staged * _LANES)],
        )

    return gather_kernel


def kernel(variable_hash, embedding_table):
    batch = variable_hash.shape[0]
    dim = embedding_table.shape[1]
    gather = _make_gather(batch, dim)
    return gather(embedding_table, variable_hash)


# speedup vs baseline: 1.5402x; 1.2511x over previous
import functools

import jax
import jax.numpy as jnp
from jax import lax
from jax.experimental import pallas as pl
from jax.experimental.pallas import tpu as pltpu
from jax.experimental.pallas import tpu_sc as plsc

_LANES = 16
_NSEM = 2


def _make_gather(batch, dim):
    info = plsc.get_sparse_core_info()
    num_workers = info.num_cores * info.num_subcores
    b_per_w = batch // num_workers
    n_bursts = b_per_w // _LANES
    mesh = plsc.VectorSubcoreMesh(core_axis_name="c", subcore_axis_name="s")

    @functools.partial(
        pl.kernel,
        mesh=mesh,
        out_type=jax.ShapeDtypeStruct((batch, dim), jnp.float32),
        scratch_types=[
            pltpu.VMEM((b_per_w,), jnp.int32),
            pltpu.VMEM((b_per_w, dim), jnp.float32),
        ]
        + [pltpu.SemaphoreType.DMA] * _NSEM,
        compiler_params=pltpu.CompilerParams(needs_layout_passes=False),
    )
    def gather_kernel(table_hbm, idx_hbm, out_hbm, idx_v, rows_v, *sems):
        wid = lax.axis_index("s") * info.num_cores + lax.axis_index("c")
        base = wid * b_per_w
        pltpu.sync_copy(idx_hbm.at[pl.ds(base, b_per_w)], idx_v)

        lane_ids = lax.iota(jnp.int32, _LANES)
        neg = jnp.full((_LANES,), jnp.iinfo(jnp.int32).min, jnp.int32)

        def burst(k):
            v = idx_v[pl.ds(k * _LANES, _LANES)]
            for l in range(_LANES):
                row = lax.reduce_max(
                    jnp.where(lane_ids == l, v, neg), axes=(0,)
                )
                pltpu.async_copy(
                    table_hbm.at[pl.ds(row, 1)],
                    rows_v.at[pl.ds(k * _LANES + l, 1)],
                    sems[l % _NSEM],
                )

        pl.loop(0, n_bursts)(burst)

        def drain(_):
            for l in range(_LANES):
                pltpu.make_async_copy(
                    table_hbm.at[pl.ds(0, 1)],
                    rows_v.at[pl.ds(0, 1)],
                    sems[l % _NSEM],
                ).wait()

        pl.loop(0, n_bursts)(drain)
        pltpu.sync_copy(rows_v, out_hbm.at[pl.ds(base, b_per_w)])

    return gather_kernel


def kernel(variable_hash, embedding_table):
    batch = variable_hash.shape[0]
    dim = embedding_table.shape[1]
    gather = _make_gather(batch, dim)
    return gather(embedding_table, variable_hash)
